# baseline (speedup 1.0000x reference)
import jax
import jax.numpy as jnp
from jax.experimental import pallas as pl

B, L, N = 16, 336, 32
PRED, D, DFF, E = 96, 64, 128, 4
PP = 128
EPS = 1e-5
KC = 2688
NKC = (L * D) // KC


def _stage_a(xT_ref, Ws_ref, bs_ref, wg_ref, We1_ref, be1_ref, W2bd_ref,
             be2_ref, e0_ref, e1_ref, e2_ref, e3_ref, gates_ref, mu_ref,
             std_ref):
    bf16 = jnp.bfloat16
    xb = xT_ref[0]
    mu = jnp.mean(xb, axis=1, keepdims=True)
    xc = xb - mu
    var = jnp.mean(xc * xc, axis=1, keepdims=True)
    std = jnp.sqrt(var + EPS)
    s = xc / std
    mu_ref[0] = jnp.transpose(mu)
    std_ref[0] = jnp.transpose(std)

    ws_bar = jnp.mean(Ws_ref[...])
    bs_bar = jnp.mean(bs_ref[...])
    xg = jnp.mean(s, axis=0, keepdims=True) * ws_bar + bs_bar
    logits = jnp.dot(xg, wg_ref[...], preferred_element_type=jnp.float32)
    iota4 = jax.lax.broadcasted_iota(jnp.int32, (1, E), 1)
    m1 = jnp.max(logits, axis=1, keepdims=True)
    i1 = jnp.min(jnp.where(logits == m1, iota4, E), axis=1, keepdims=True)
    l2 = jnp.where(iota4 == i1, -3e38, logits)
    m2 = jnp.max(l2, axis=1, keepdims=True)
    i2 = jnp.min(jnp.where(l2 == m2, iota4, E), axis=1, keepdims=True)
    e2v = jnp.exp(m2 - m1)
    den = 1.0 + e2v
    gates_v = (jnp.where(iota4 == i1, 1.0 / den, 0.0)
               + jnp.where(iota4 == i2, e2v / den, 0.0))
    gates_ref[0] = gates_v

    ws_row = Ws_ref[...]
    us = [jnp.dot(ws_row, We1_ref[i], preferred_element_type=jnp.float32)
          for i in range(E)]
    vs = [jnp.dot(bs_ref[...], We1_ref[i], preferred_element_type=jnp.float32)
          + be1_ref[i:i + 1] for i in range(E)]
    u_all = jnp.concatenate(us, axis=1).astype(bf16)
    v_all = jnp.concatenate(vs, axis=1).astype(bf16)
    s_bf = s.astype(bf16)
    h3 = (s_bf[:, :, None] * ws_row.astype(bf16)[None]
          + bs_ref[...].astype(bf16)[None])
    a3 = s_bf[:, :, None] * u_all[None] + v_all[None]
    g3 = jax.nn.gelu(a3, approximate=True)
    g2 = g3.reshape(N * L, E * DFF)
    m_all = jnp.dot(g2, W2bd_ref[...],
                    preferred_element_type=jnp.float32)
    m3 = m_all.astype(bf16).reshape(N, L, E * D)
    e_refs = (e0_ref, e1_ref, e2_ref, e3_ref)
    for i in range(E):
        mi = m3[:, :, i * D:(i + 1) * D]
        e3 = mi + be2_ref[i:i + 1][None].astype(bf16) + h3
        e_refs[i][0] = e3


def _stage_b(ef0_ref, ef1_ref, ef2_ref, ef3_ref, wp_ref, wep_ref, grow_ref,
             mu_ref, std_ref, bpp_ref, bepp_ref, y_ref, ey_ref):
    bf16 = jnp.bfloat16
    ef_refs = (ef0_ref, ef1_ref, ef2_ref, ef3_ref)
    kc = pl.program_id(0)
    of = None
    for i in range(E):
        eb = ef_refs[i][...]
        gcol = jnp.transpose(grow_ref[i:i + 1]).astype(bf16)
        contrib = eb * gcol
        of = contrib if of is None else of + contrib
        pey = jnp.dot(eb, wep_ref[i].astype(bf16),
                      preferred_element_type=jnp.float32)

        @pl.when(kc == 0)
        def _(i=i, pey=pey):
            ey_ref[i] = pey

        @pl.when(kc > 0)
        def _(i=i, pey=pey):
            ey_ref[i] = ey_ref[i] + pey

    py = jnp.dot(of, wp_ref[...].astype(bf16),
                 preferred_element_type=jnp.float32)

    @pl.when(kc == 0)
    def _():
        y_ref[...] = py

    @pl.when(kc > 0)
    def _():
        y_ref[...] = y_ref[...] + py

    @pl.when(kc == NKC - 1)
    def _():
        std_c = jnp.transpose(std_ref[...])
        mu_c = jnp.transpose(mu_ref[...])
        y_ref[...] = (y_ref[...] + bpp_ref[...]) * std_c + mu_c
        for i in range(E):
            ey_ref[i] = (ey_ref[i] + bepp_ref[i:i + 1]) * std_c + mu_c


def kernel(x, Ws, bs, w_gate, We1, be1, We2, be2, Wp, bp, Wep, bep):
    f32 = jnp.float32
    bf16 = jnp.bfloat16
    xT = jnp.transpose(x, (0, 2, 1))
    bs2 = bs.reshape(1, D)
    w2bd = jnp.zeros((E * DFF, E * D), f32)
    for i in range(E):
        w2bd = w2bd.at[i * DFF:(i + 1) * DFF, i * D:(i + 1) * D].set(We2[i])
    w2bd = w2bd.astype(bf16)
    bpp = bp.reshape(1, PRED)
    bepp = bep

    e_o0, e_o1, e_o2, e_o3, gates_o, mu_o, std_o = pl.pallas_call(
        _stage_a,
        grid=(B,),
        in_specs=[
            pl.BlockSpec((1, N, L), lambda b: (b, 0, 0)),
            pl.BlockSpec((1, D), lambda b: (0, 0)),
            pl.BlockSpec((1, D), lambda b: (0, 0)),
            pl.BlockSpec((L, E), lambda b: (0, 0)),
            pl.BlockSpec((E, D, DFF), lambda b: (0, 0, 0)),
            pl.BlockSpec((E, DFF), lambda b: (0, 0)),
            pl.BlockSpec((E * DFF, E * D), lambda b: (0, 0)),
            pl.BlockSpec((E, D), lambda b: (0, 0)),
        ],
        out_specs=[
            pl.BlockSpec((1, N, L, D), lambda b: (b, 0, 0, 0)),
            pl.BlockSpec((1, N, L, D), lambda b: (b, 0, 0, 0)),
            pl.BlockSpec((1, N, L, D), lambda b: (b, 0, 0, 0)),
            pl.BlockSpec((1, N, L, D), lambda b: (b, 0, 0, 0)),
            pl.BlockSpec((1, 1, E), lambda b: (b, 0, 0)),
            pl.BlockSpec((1, 1, N), lambda b: (b, 0, 0)),
            pl.BlockSpec((1, 1, N), lambda b: (b, 0, 0)),
        ],
        out_shape=[
            jax.ShapeDtypeStruct((B, N, L, D), bf16),
            jax.ShapeDtypeStruct((B, N, L, D), bf16),
            jax.ShapeDtypeStruct((B, N, L, D), bf16),
            jax.ShapeDtypeStruct((B, N, L, D), bf16),
            jax.ShapeDtypeStruct((B, 1, E), f32),
            jax.ShapeDtypeStruct((B, 1, N), f32),
            jax.ShapeDtypeStruct((B, 1, N), f32),
        ],
    )(xT, Ws, bs2, w_gate, We1, be1, w2bd, be2)

    eflats = [e.reshape(B * N, L * D) for e in (e_o0, e_o1, e_o2, e_o3)]
    g2 = gates_o[:, 0, :]
    grow = jnp.broadcast_to(
        jnp.transpose(g2)[:, :, None], (E, B, N)
    ).reshape(E, B * N)
    mu_r = mu_o.reshape(1, B * N)
    std_r = std_o.reshape(1, B * N)

    y2, ey2 = pl.pallas_call(
        _stage_b,
        grid=(NKC,),
        in_specs=[
            pl.BlockSpec((B * N, KC), lambda k: (0, k)),
            pl.BlockSpec((B * N, KC), lambda k: (0, k)),
            pl.BlockSpec((B * N, KC), lambda k: (0, k)),
            pl.BlockSpec((B * N, KC), lambda k: (0, k)),
            pl.BlockSpec((KC, PRED), lambda k: (k, 0)),
            pl.BlockSpec((E, KC, PRED), lambda k: (0, k, 0)),
            pl.BlockSpec((E, B * N), lambda k: (0, 0)),
            pl.BlockSpec((1, B * N), lambda k: (0, 0)),
            pl.BlockSpec((1, B * N), lambda k: (0, 0)),
            pl.BlockSpec((1, PRED), lambda k: (0, 0)),
            pl.BlockSpec((E, PRED), lambda k: (0, 0)),
        ],
        out_specs=[
            pl.BlockSpec((B * N, PRED), lambda k: (0, 0)),
            pl.BlockSpec((E, B * N, PRED), lambda k: (0, 0, 0)),
        ],
        out_shape=[
            jax.ShapeDtypeStruct((B * N, PRED), f32),
            jax.ShapeDtypeStruct((E, B * N, PRED), f32),
        ],
    )(eflats[0], eflats[1], eflats[2], eflats[3], Wp, Wep, grow,
      mu_r, std_r, bpp, bepp)

    y = jnp.transpose(y2.reshape(B, N, PRED), (0, 2, 1))
    ey = jnp.transpose(ey2.reshape(E, B, N, PRED), (0, 1, 3, 2))
    return (y, ey)


# speedup vs baseline: 1.4698x; 1.4698x over previous
import jax
import jax.numpy as jnp
from jax.experimental import pallas as pl

B, L, N = 16, 336, 32
PRED, D, DFF, E = 96, 64, 128, 4
PP = 128
EPS = 1e-5
KC = 2688
NKC = (L * D) // KC


def _stage_a(xT_ref, Ws_ref, bs_ref, wg_ref, We1_ref, be1_ref, W2bd_ref,
             be2_ref, e0_ref, e1_ref, e2_ref, e3_ref, gates_ref, mu_ref,
             std_ref):
    bf16 = jnp.bfloat16
    xb = xT_ref[0]
    mu = jnp.mean(xb, axis=1, keepdims=True)
    xc = xb - mu
    var = jnp.mean(xc * xc, axis=1, keepdims=True)
    std = jnp.sqrt(var + EPS)
    s = xc / std
    mu_ref[0] = jnp.transpose(mu)
    std_ref[0] = jnp.transpose(std)

    ws_bar = jnp.mean(Ws_ref[...])
    bs_bar = jnp.mean(bs_ref[...])
    xg = jnp.mean(s, axis=0, keepdims=True) * ws_bar + bs_bar
    logits = jnp.dot(xg, wg_ref[...], preferred_element_type=jnp.float32)
    iota4 = jax.lax.broadcasted_iota(jnp.int32, (1, E), 1)
    m1 = jnp.max(logits, axis=1, keepdims=True)
    i1 = jnp.min(jnp.where(logits == m1, iota4, E), axis=1, keepdims=True)
    l2 = jnp.where(iota4 == i1, -3e38, logits)
    m2 = jnp.max(l2, axis=1, keepdims=True)
    i2 = jnp.min(jnp.where(l2 == m2, iota4, E), axis=1, keepdims=True)
    e2v = jnp.exp(m2 - m1)
    den = 1.0 + e2v
    gates_v = (jnp.where(iota4 == i1, 1.0 / den, 0.0)
               + jnp.where(iota4 == i2, e2v / den, 0.0))
    gates_ref[0] = gates_v

    ws_row = Ws_ref[...]
    us = [jnp.dot(ws_row, We1_ref[i], preferred_element_type=jnp.float32)
          for i in range(E)]
    vs = [jnp.dot(bs_ref[...], We1_ref[i], preferred_element_type=jnp.float32)
          + be1_ref[i:i + 1] for i in range(E)]
    u_all = jnp.concatenate(us, axis=1).astype(bf16)
    v_all = jnp.concatenate(vs, axis=1).astype(bf16)
    s_bf = s.astype(bf16)
    h3 = (s_bf[:, :, None] * ws_row.astype(bf16)[None]
          + bs_ref[...].astype(bf16)[None])
    a3 = s_bf[:, :, None] * u_all[None] + v_all[None]
    g3 = jax.nn.gelu(a3, approximate=True)
    g2 = g3.reshape(N * L, E * DFF)
    m_all = jnp.dot(g2, W2bd_ref[...],
                    preferred_element_type=jnp.float32)
    m3 = m_all.astype(bf16).reshape(N, L, E * D)
    e_refs = (e0_ref, e1_ref, e2_ref, e3_ref)
    for i in range(E):
        mi = m3[:, :, i * D:(i + 1) * D]
        e3 = mi + be2_ref[i:i + 1][None].astype(bf16) + h3
        e_refs[i][0] = jnp.concatenate(
            [e3[:, :L // 2, :], e3[:, L // 2:, :]], axis=2)


def _interleave(wlo, whi):
    w3 = jnp.concatenate([wlo.reshape(KC // (2 * D), D, PRED),
                          whi.reshape(KC // (2 * D), D, PRED)], axis=1)
    return w3.reshape(KC, PRED).astype(jnp.bfloat16)


def _stage_b(ef0_ref, ef1_ref, ef2_ref, ef3_ref, wplo_ref, wphi_ref,
             weplo_ref, wephi_ref, grow_ref,
             mu_ref, std_ref, bpp_ref, bepp_ref, y_ref, ey_ref):
    bf16 = jnp.bfloat16
    ef_refs = (ef0_ref, ef1_ref, ef2_ref, ef3_ref)
    kc = pl.program_id(0)
    of = None
    for i in range(E):
        eb = ef_refs[i][...]
        gcol = jnp.transpose(grow_ref[i:i + 1]).astype(bf16)
        contrib = eb * gcol
        of = contrib if of is None else of + contrib
        wepb = _interleave(weplo_ref[i, 0], wephi_ref[i, 0])
        pey = jnp.dot(eb, wepb, preferred_element_type=jnp.float32)

        @pl.when(kc == 0)
        def _(i=i, pey=pey):
            ey_ref[i] = pey

        @pl.when(kc > 0)
        def _(i=i, pey=pey):
            ey_ref[i] = ey_ref[i] + pey

    wpb = _interleave(wplo_ref[0], wphi_ref[0])
    py = jnp.dot(of, wpb, preferred_element_type=jnp.float32)

    @pl.when(kc == 0)
    def _():
        y_ref[...] = py

    @pl.when(kc > 0)
    def _():
        y_ref[...] = y_ref[...] + py

    @pl.when(kc == NKC - 1)
    def _():
        std_c = jnp.transpose(std_ref[...])
        mu_c = jnp.transpose(mu_ref[...])
        y_ref[...] = (y_ref[...] + bpp_ref[...]) * std_c + mu_c
        for i in range(E):
            ey_ref[i] = (ey_ref[i] + bepp_ref[i:i + 1]) * std_c + mu_c


def kernel(x, Ws, bs, w_gate, We1, be1, We2, be2, Wp, bp, Wep, bep):
    f32 = jnp.float32
    bf16 = jnp.bfloat16
    xT = jnp.transpose(x, (0, 2, 1))
    bs2 = bs.reshape(1, D)
    w2bd = jnp.zeros((E * DFF, E * D), f32)
    for i in range(E):
        w2bd = w2bd.at[i * DFF:(i + 1) * DFF, i * D:(i + 1) * D].set(We2[i])
    w2bd = w2bd.astype(bf16)
    wp2 = Wp.reshape(2, (L // 2) * D, PRED)
    wep2 = Wep.reshape(E, 2, (L // 2) * D, PRED)
    bpp = bp.reshape(1, PRED)
    bepp = bep

    e_o0, e_o1, e_o2, e_o3, gates_o, mu_o, std_o = pl.pallas_call(
        _stage_a,
        grid=(B,),
        in_specs=[
            pl.BlockSpec((1, N, L), lambda b: (b, 0, 0)),
            pl.BlockSpec((1, D), lambda b: (0, 0)),
            pl.BlockSpec((1, D), lambda b: (0, 0)),
            pl.BlockSpec((L, E), lambda b: (0, 0)),
            pl.BlockSpec((E, D, DFF), lambda b: (0, 0, 0)),
            pl.BlockSpec((E, DFF), lambda b: (0, 0)),
            pl.BlockSpec((E * DFF, E * D), lambda b: (0, 0)),
            pl.BlockSpec((E, D), lambda b: (0, 0)),
        ],
        out_specs=[
            pl.BlockSpec((1, N, L // 2, 2 * D), lambda b: (b, 0, 0, 0)),
            pl.BlockSpec((1, N, L // 2, 2 * D), lambda b: (b, 0, 0, 0)),
            pl.BlockSpec((1, N, L // 2, 2 * D), lambda b: (b, 0, 0, 0)),
            pl.BlockSpec((1, N, L // 2, 2 * D), lambda b: (b, 0, 0, 0)),
            pl.BlockSpec((1, 1, E), lambda b: (b, 0, 0)),
            pl.BlockSpec((1, 1, N), lambda b: (b, 0, 0)),
            pl.BlockSpec((1, 1, N), lambda b: (b, 0, 0)),
        ],
        out_shape=[
            jax.ShapeDtypeStruct((B, N, L // 2, 2 * D), bf16),
            jax.ShapeDtypeStruct((B, N, L // 2, 2 * D), bf16),
            jax.ShapeDtypeStruct((B, N, L // 2, 2 * D), bf16),
            jax.ShapeDtypeStruct((B, N, L // 2, 2 * D), bf16),
            jax.ShapeDtypeStruct((B, 1, E), f32),
            jax.ShapeDtypeStruct((B, 1, N), f32),
            jax.ShapeDtypeStruct((B, 1, N), f32),
        ],
    )(xT, Ws, bs2, w_gate, We1, be1, w2bd, be2)

    eflats = [e.reshape(B * N, L * D) for e in (e_o0, e_o1, e_o2, e_o3)]
    g2 = gates_o[:, 0, :]
    grow = jnp.broadcast_to(
        jnp.transpose(g2)[:, :, None], (E, B, N)
    ).reshape(E, B * N)
    mu_r = mu_o.reshape(1, B * N)
    std_r = std_o.reshape(1, B * N)

    y2, ey2 = pl.pallas_call(
        _stage_b,
        grid=(NKC,),
        in_specs=[
            pl.BlockSpec((B * N, KC), lambda k: (0, k)),
            pl.BlockSpec((B * N, KC), lambda k: (0, k)),
            pl.BlockSpec((B * N, KC), lambda k: (0, k)),
            pl.BlockSpec((B * N, KC), lambda k: (0, k)),
            pl.BlockSpec((1, KC // 2, PRED), lambda k: (0, k, 0)),
            pl.BlockSpec((1, KC // 2, PRED), lambda k: (1, k, 0)),
            pl.BlockSpec((E, 1, KC // 2, PRED), lambda k: (0, 0, k, 0)),
            pl.BlockSpec((E, 1, KC // 2, PRED), lambda k: (0, 1, k, 0)),
            pl.BlockSpec((E, B * N), lambda k: (0, 0)),
            pl.BlockSpec((1, B * N), lambda k: (0, 0)),
            pl.BlockSpec((1, B * N), lambda k: (0, 0)),
            pl.BlockSpec((1, PRED), lambda k: (0, 0)),
            pl.BlockSpec((E, PRED), lambda k: (0, 0)),
        ],
        out_specs=[
            pl.BlockSpec((B * N, PRED), lambda k: (0, 0)),
            pl.BlockSpec((E, B * N, PRED), lambda k: (0, 0, 0)),
        ],
        out_shape=[
            jax.ShapeDtypeStruct((B * N, PRED), f32),
            jax.ShapeDtypeStruct((E, B * N, PRED), f32),
        ],
    )(eflats[0], eflats[1], eflats[2], eflats[3], wp2, wp2, wep2, wep2,
      grow, mu_r, std_r, bpp, bepp)

    y = jnp.transpose(y2.reshape(B, N, PRED), (0, 2, 1))
    ey = jnp.transpose(ey2.reshape(E, B, N, PRED), (0, 1, 3, 2))
    return (y, ey)


# speedup vs baseline: 1.4988x; 1.0197x over previous
import jax
import jax.numpy as jnp
from jax.experimental import pallas as pl

B, L, N = 16, 336, 32
PRED, D, DFF, E = 96, 64, 128, 4
PP = 128
EPS = 1e-5
KC = 2688
NKC = (L * D) // KC


def _stage_a(xT_ref, Ws_ref, bs_ref, wg_ref, We1_ref, be1_ref, W2bd_ref,
             be2_ref, e0_ref, e1_ref, e2_ref, e3_ref, gates_ref, mu_ref,
             std_ref):
    bf16 = jnp.bfloat16
    xb = xT_ref[0]
    mu = jnp.mean(xb, axis=1, keepdims=True)
    xc = xb - mu
    var = jnp.mean(xc * xc, axis=1, keepdims=True)
    std = jnp.sqrt(var + EPS)
    s = xc / std
    mu_ref[0] = jnp.transpose(mu)
    std_ref[0] = jnp.transpose(std)

    ws_bar = jnp.mean(Ws_ref[...])
    bs_bar = jnp.mean(bs_ref[...])
    xg = jnp.mean(s, axis=0, keepdims=True) * ws_bar + bs_bar
    logits = jnp.dot(xg, wg_ref[...], preferred_element_type=jnp.float32)
    iota4 = jax.lax.broadcasted_iota(jnp.int32, (1, E), 1)
    m1 = jnp.max(logits, axis=1, keepdims=True)
    i1 = jnp.min(jnp.where(logits == m1, iota4, E), axis=1, keepdims=True)
    l2 = jnp.where(iota4 == i1, -3e38, logits)
    m2 = jnp.max(l2, axis=1, keepdims=True)
    i2 = jnp.min(jnp.where(l2 == m2, iota4, E), axis=1, keepdims=True)
    e2v = jnp.exp(m2 - m1)
    den = 1.0 + e2v
    gates_v = (jnp.where(iota4 == i1, 1.0 / den, 0.0)
               + jnp.where(iota4 == i2, e2v / den, 0.0))
    gates_ref[0] = gates_v

    ws_row = Ws_ref[...]
    us = [jnp.dot(ws_row, We1_ref[i], preferred_element_type=jnp.float32)
          for i in range(E)]
    vs = [jnp.dot(bs_ref[...], We1_ref[i], preferred_element_type=jnp.float32)
          + be1_ref[i:i + 1] for i in range(E)]
    u_all = jnp.concatenate(us, axis=1).astype(bf16)
    v_all = jnp.concatenate(vs, axis=1).astype(bf16)
    s_bf = s.astype(bf16)
    h3 = (s_bf[:, :, None] * ws_row.astype(bf16)[None]
          + bs_ref[...].astype(bf16)[None])
    a3 = s_bf[:, :, None] * u_all[None] + v_all[None]
    g3 = jax.nn.gelu(a3, approximate=True)
    g2 = g3.reshape(N * L, E * DFF)
    m_all = jnp.dot(g2, W2bd_ref[...],
                    preferred_element_type=jnp.float32)
    m3 = m_all.astype(bf16).reshape(N, L, E * D)
    e_refs = (e0_ref, e1_ref, e2_ref, e3_ref)
    for i in range(E):
        mi = m3[:, :, i * D:(i + 1) * D]
        e3 = mi + be2_ref[i:i + 1][None].astype(bf16) + h3
        e_refs[i][0] = jnp.concatenate(
            [e3[:, :L // 2, :], e3[:, L // 2:, :]], axis=2)


def _interleave3(wlo, whi):
    return jnp.concatenate([wlo.reshape(KC // (2 * D), D, PRED),
                            whi.reshape(KC // (2 * D), D, PRED)], axis=1)


def _stage_b(ef0_ref, ef1_ref, ef2_ref, ef3_ref, wplo_ref, wphi_ref,
             weplo_ref, wephi_ref, grow_ref,
             mu_ref, std_ref, bpp_ref, bepp_ref, y_ref, ey_ref):
    bf16 = jnp.bfloat16
    ef_refs = (ef0_ref, ef1_ref, ef2_ref, ef3_ref)
    kc = pl.program_id(0)
    wp3 = _interleave3(wplo_ref[0], wphi_ref[0])
    ysum = None
    for i in range(E):
        eb = ef_refs[i][...]
        wep3 = _interleave3(weplo_ref[i, 0], wephi_ref[i, 0])
        wcat = jnp.concatenate([wep3, wp3], axis=2).reshape(
            KC, 2 * PRED).astype(bf16)
        pcat = jnp.dot(eb, wcat, preferred_element_type=jnp.float32)
        pey = pcat[:, :PRED]
        gcol = jnp.transpose(grow_ref[i:i + 1])
        pyi = pcat[:, PRED:] * gcol
        ysum = pyi if ysum is None else ysum + pyi

        @pl.when(kc == 0)
        def _(i=i, pey=pey):
            ey_ref[i] = pey

        @pl.when(kc > 0)
        def _(i=i, pey=pey):
            ey_ref[i] = ey_ref[i] + pey

    @pl.when(kc == 0)
    def _():
        y_ref[...] = ysum

    @pl.when(kc > 0)
    def _():
        y_ref[...] = y_ref[...] + ysum

    @pl.when(kc == NKC - 1)
    def _():
        std_c = jnp.transpose(std_ref[...])
        mu_c = jnp.transpose(mu_ref[...])
        y_ref[...] = (y_ref[...] + bpp_ref[...]) * std_c + mu_c
        for i in range(E):
            ey_ref[i] = (ey_ref[i] + bepp_ref[i:i + 1]) * std_c + mu_c


def kernel(x, Ws, bs, w_gate, We1, be1, We2, be2, Wp, bp, Wep, bep):
    f32 = jnp.float32
    bf16 = jnp.bfloat16
    xT = jnp.transpose(x, (0, 2, 1))
    bs2 = bs.reshape(1, D)
    w2bd = jnp.zeros((E * DFF, E * D), f32)
    for i in range(E):
        w2bd = w2bd.at[i * DFF:(i + 1) * DFF, i * D:(i + 1) * D].set(We2[i])
    w2bd = w2bd.astype(bf16)
    wp2 = Wp.reshape(2, (L // 2) * D, PRED)
    wep2 = Wep.reshape(E, 2, (L // 2) * D, PRED)
    bpp = bp.reshape(1, PRED)
    bepp = bep

    e_o0, e_o1, e_o2, e_o3, gates_o, mu_o, std_o = pl.pallas_call(
        _stage_a,
        grid=(B,),
        in_specs=[
            pl.BlockSpec((1, N, L), lambda b: (b, 0, 0)),
            pl.BlockSpec((1, D), lambda b: (0, 0)),
            pl.BlockSpec((1, D), lambda b: (0, 0)),
            pl.BlockSpec((L, E), lambda b: (0, 0)),
            pl.BlockSpec((E, D, DFF), lambda b: (0, 0, 0)),
            pl.BlockSpec((E, DFF), lambda b: (0, 0)),
            pl.BlockSpec((E * DFF, E * D), lambda b: (0, 0)),
            pl.BlockSpec((E, D), lambda b: (0, 0)),
        ],
        out_specs=[
            pl.BlockSpec((1, N, L // 2, 2 * D), lambda b: (b, 0, 0, 0)),
            pl.BlockSpec((1, N, L // 2, 2 * D), lambda b: (b, 0, 0, 0)),
            pl.BlockSpec((1, N, L // 2, 2 * D), lambda b: (b, 0, 0, 0)),
            pl.BlockSpec((1, N, L // 2, 2 * D), lambda b: (b, 0, 0, 0)),
            pl.BlockSpec((1, 1, E), lambda b: (b, 0, 0)),
            pl.BlockSpec((1, 1, N), lambda b: (b, 0, 0)),
            pl.BlockSpec((1, 1, N), lambda b: (b, 0, 0)),
        ],
        out_shape=[
            jax.ShapeDtypeStruct((B, N, L // 2, 2 * D), bf16),
            jax.ShapeDtypeStruct((B, N, L // 2, 2 * D), bf16),
            jax.ShapeDtypeStruct((B, N, L // 2, 2 * D), bf16),
            jax.ShapeDtypeStruct((B, N, L // 2, 2 * D), bf16),
            jax.ShapeDtypeStruct((B, 1, E), f32),
            jax.ShapeDtypeStruct((B, 1, N), f32),
            jax.ShapeDtypeStruct((B, 1, N), f32),
        ],
    )(xT, Ws, bs2, w_gate, We1, be1, w2bd, be2)

    eflats = [e.reshape(B * N, L * D) for e in (e_o0, e_o1, e_o2, e_o3)]
    g2 = gates_o[:, 0, :]
    grow = jnp.broadcast_to(
        jnp.transpose(g2)[:, :, None], (E, B, N)
    ).reshape(E, B * N)
    mu_r = mu_o.reshape(1, B * N)
    std_r = std_o.reshape(1, B * N)

    y2, ey2 = pl.pallas_call(
        _stage_b,
        grid=(NKC,),
        in_specs=[
            pl.BlockSpec((B * N, KC), lambda k: (0, k)),
            pl.BlockSpec((B * N, KC), lambda k: (0, k)),
            pl.BlockSpec((B * N, KC), lambda k: (0, k)),
            pl.BlockSpec((B * N, KC), lambda k: (0, k)),
            pl.BlockSpec((1, KC // 2, PRED), lambda k: (0, k, 0)),
            pl.BlockSpec((1, KC // 2, PRED), lambda k: (1, k, 0)),
            pl.BlockSpec((E, 1, KC // 2, PRED), lambda k: (0, 0, k, 0)),
            pl.BlockSpec((E, 1, KC // 2, PRED), lambda k: (0, 1, k, 0)),
            pl.BlockSpec((E, B * N), lambda k: (0, 0)),
            pl.BlockSpec((1, B * N), lambda k: (0, 0)),
            pl.BlockSpec((1, B * N), lambda k: (0, 0)),
            pl.BlockSpec((1, PRED), lambda k: (0, 0)),
            pl.BlockSpec((E, PRED), lambda k: (0, 0)),
        ],
        out_specs=[
            pl.BlockSpec((B * N, PRED), lambda k: (0, 0)),
            pl.BlockSpec((E, B * N, PRED), lambda k: (0, 0, 0)),
        ],
        out_shape=[
            jax.ShapeDtypeStruct((B * N, PRED), f32),
            jax.ShapeDtypeStruct((E, B * N, PRED), f32),
        ],
    )(eflats[0], eflats[1], eflats[2], eflats[3], wp2, wp2, wep2, wep2,
      grow, mu_r, std_r, bpp, bepp)

    y = jnp.transpose(y2.reshape(B, N, PRED), (0, 2, 1))
    ey = jnp.transpose(ey2.reshape(E, B, N, PRED), (0, 1, 3, 2))
    return (y, ey)
